# Initial kernel scaffold; baseline (speedup 1.0000x reference)
import jax
import jax.numpy as jnp
from jax.experimental import pallas as pl


def kernel(x, edge_index, W1, b1, gamma, beta, W2, b2):
    raise NotImplementedError("write your pallas kernel here")


# speedup vs baseline: 3.3421x; 3.3421x over previous
import functools

import jax
import jax.numpy as jnp
from jax import lax
from jax.experimental import pallas as pl
from jax.experimental.pallas import tpu as pltpu
from jax.experimental.pallas import tpu_sc as plsc

EPS = 1e-07
NC = 2
NS = 16
LANES = 16
CH = 128


def _sc_edge_pass(n_nodes, d_feat, e_pad):
    half = d_feat // 2
    epw = e_pad // NS
    nchunk = epw // CH
    nacc = n_nodes + LANES
    rpw = (n_nodes // NS) & ~7
    tail = n_nodes - NS * rpw
    ztail = nacc - NS * rpw
    mesh = plsc.VectorSubcoreMesh(core_axis_name="c", subcore_axis_name="s")

    def _chunked(total):
        done = 0
        while done < total:
            step = min(CH, total - done)
            yield done, step
            done += step

    @functools.partial(
        pl.kernel,
        out_type=jax.ShapeDtypeStruct((NC, n_nodes, d_feat), jnp.float32),
        mesh=mesh,
        compiler_params=pltpu.CompilerParams(use_tc_tiling_on_sc=False),
        scratch_types=[
            pltpu.VMEM((CH,), jnp.int32),
            pltpu.VMEM((CH,), jnp.int32),
            pltpu.VMEM((CH,), jnp.int32),
            pltpu.VMEM((CH, half), jnp.float32),
            pltpu.VMEM((CH, d_feat), jnp.float32),
            pltpu.VMEM_SHARED((nacc, d_feat), jnp.float32),
            pltpu.SemaphoreType.DMA,
        ],
    )
    def sc_kernel(xr, src, dst, out, src_v, dst_v, idx_v, gbuf, sbuf, acc,
                  gsem):
        c = lax.axis_index("c")
        s = lax.axis_index("s")

        def zrow(r, _):
            for j in range(d_feat // LANES):
                sbuf[r, pl.ds(j * LANES, LANES)] = jnp.zeros(
                    (LANES,), jnp.float32)
            return 0
        lax.fori_loop(0, CH, zrow, 0)
        zbase = s * rpw
        for off, step in _chunked(rpw):
            pltpu.sync_copy(sbuf.at[pl.ds(0, step)],
                            acc.at[pl.ds(zbase + off, step)])
        if ztail:
            @pl.when(s == NS - 1)
            def _():
                for off, step in _chunked(ztail):
                    pltpu.sync_copy(
                        sbuf.at[pl.ds(0, step)],
                        acc.at[pl.ds(NS * rpw + off, step)])
        plsc.subcore_barrier()

        def chunk_body(k, _):
            base = s * epw + k * CH
            pltpu.sync_copy(src.at[pl.ds(base, CH)], src_v)
            pltpu.sync_copy(dst.at[pl.ds(base, CH)], dst_v)
            for i in range(CH // LANES):
                v = src_v[pl.ds(i * LANES, LANES)]
                idx_v[pl.ds(i * LANES, LANES)] = v * 2 + c
            pltpu.async_copy(xr.at[idx_v], gbuf, gsem).wait()

            def edge_body(e, _):
                for j in range(half // LANES):
                    v = gbuf[e, pl.ds(j * LANES, LANES)]
                    g = jnp.maximum(v, 0.0) + EPS
                    ex = jnp.exp(g)
                    sbuf[e, pl.ds(j * LANES, LANES)] = ex
                    sbuf[e, pl.ds(half + j * LANES, LANES)] = g * ex
                return 0
            lax.fori_loop(0, CH, edge_body, 0)
            pltpu.sync_copy(sbuf, acc.at[dst_v], add=True)
            return 0
        lax.fori_loop(0, nchunk, chunk_body, 0)
        plsc.subcore_barrier()

        obase = s * rpw
        for off, step in _chunked(rpw):
            pltpu.sync_copy(acc.at[pl.ds(obase + off, step)],
                            out.at[c, pl.ds(obase + off, step)])
        if tail:
            @pl.when(s == NS - 1)
            def _():
                for off, step in _chunked(tail):
                    pltpu.sync_copy(
                        acc.at[pl.ds(NS * rpw + off, step)],
                        out.at[c, pl.ds(NS * rpw + off, step)])

    return sc_kernel


def _tc_mlp1(n_nodes, d_feat, h_feat, blk):
    half = d_feat // 2
    grid = n_nodes // blk

    def body(x_ref, s_ref, w1_ref, b1_ref, h1_ref, sums_ref):
        s0 = s_ref[0]
        s1 = s_ref[1]
        ex = jnp.concatenate([s0[:, :half], s1[:, :half]], axis=1)
        gex = jnp.concatenate([s0[:, half:], s1[:, half:]], axis=1)
        m = gex / (ex + 1e-16)
        h = x_ref[...] + m
        h1 = jnp.dot(h, w1_ref[...],
                     preferred_element_type=jnp.float32) + b1_ref[...]
        h1_ref[...] = h1

        @pl.when(pl.program_id(0) == 0)
        def _():
            sums_ref[...] = jnp.zeros_like(sums_ref)

        upd = jnp.concatenate(
            [jnp.sum(h1, axis=0, keepdims=True),
             jnp.sum(h1 * h1, axis=0, keepdims=True),
             jnp.zeros((6, h_feat), jnp.float32)], axis=0)
        sums_ref[...] += upd

    return pl.pallas_call(
        body,
        grid=(grid,),
        in_specs=[
            pl.BlockSpec((blk, d_feat), lambda i: (i, 0)),
            pl.BlockSpec((NC, blk, d_feat), lambda i: (0, i, 0)),
            pl.BlockSpec((d_feat, h_feat), lambda i: (0, 0)),
            pl.BlockSpec((1, h_feat), lambda i: (0, 0)),
        ],
        out_specs=[
            pl.BlockSpec((blk, h_feat), lambda i: (i, 0)),
            pl.BlockSpec((8, h_feat), lambda i: (0, 0)),
        ],
        out_shape=[
            jax.ShapeDtypeStruct((n_nodes, h_feat), jnp.float32),
            jax.ShapeDtypeStruct((8, h_feat), jnp.float32),
        ],
    )


def _tc_mlp2(n_nodes, d_feat, h_feat, blk):
    grid = n_nodes // blk
    inv_n = 1.0 / n_nodes

    def body(h1_ref, sums_ref, gamma_ref, beta_ref, w2_ref, b2_ref, o_ref):
        mean = sums_ref[0:1, :] * inv_n
        var = sums_ref[1:2, :] * inv_n - mean * mean
        scale = lax.rsqrt(var + 1e-05) * gamma_ref[...]
        h1n = (h1_ref[...] - mean) * scale + beta_ref[...]
        h1n = jnp.maximum(h1n, 0.0)
        o_ref[...] = jnp.dot(h1n, w2_ref[...],
                             preferred_element_type=jnp.float32) + b2_ref[...]

    return pl.pallas_call(
        body,
        grid=(grid,),
        in_specs=[
            pl.BlockSpec((blk, h_feat), lambda i: (i, 0)),
            pl.BlockSpec((8, h_feat), lambda i: (0, 0)),
            pl.BlockSpec((1, h_feat), lambda i: (0, 0)),
            pl.BlockSpec((1, h_feat), lambda i: (0, 0)),
            pl.BlockSpec((h_feat, d_feat), lambda i: (0, 0)),
            pl.BlockSpec((1, d_feat), lambda i: (0, 0)),
        ],
        out_specs=pl.BlockSpec((blk, d_feat), lambda i: (i, 0)),
        out_shape=jax.ShapeDtypeStruct((n_nodes, d_feat), jnp.float32),
    )


def kernel(x, edge_index, W1, b1, gamma, beta, W2, b2):
    n, d = x.shape
    h_feat = W1.shape[1]
    e = edge_index.shape[1]

    e_pad = ((e + NS * CH - 1) // (NS * CH)) * (NS * CH)
    pad = e_pad - e
    src = edge_index[0]
    dst = edge_index[1]
    if pad:
        src = jnp.concatenate([src, jnp.zeros((pad,), jnp.int32)])
        dst = jnp.concatenate([dst, jnp.full((pad,), n, jnp.int32)])
    xr = x.reshape(2 * n, d // 2)

    s_acc = _sc_edge_pass(n, d, e_pad)(xr, src, dst)

    blk = 1000 if n % 1000 == 0 else n // 8
    h1, sums = _tc_mlp1(n, d, h_feat, blk)(
        x, s_acc, W1, b1.reshape(1, h_feat))
    out = _tc_mlp2(n, d, h_feat, blk)(
        h1, sums, gamma.reshape(1, h_feat), beta.reshape(1, h_feat),
        W2, b2.reshape(1, d))
    return out


# speedup vs baseline: 10.8604x; 3.2495x over previous
import functools

import jax
import jax.numpy as jnp
from jax import lax
from jax.experimental import pallas as pl
from jax.experimental.pallas import tpu as pltpu
from jax.experimental.pallas import tpu_sc as plsc

EPS = 1e-07
NC = 2
NS = 16
LANES = 16
CH = 128


def _sc_edge_pass(n_nodes, d_feat, e_pad):
    half = d_feat // 2
    epw = e_pad // NS
    nchunk = epw // CH
    nacc = n_nodes + LANES
    rpw = (n_nodes // NS) & ~7
    tail = n_nodes - NS * rpw
    ztail = nacc - NS * rpw
    mesh = plsc.VectorSubcoreMesh(core_axis_name="c", subcore_axis_name="s")

    def _chunked(total):
        done = 0
        while done < total:
            step = min(CH, total - done)
            yield done, step
            done += step

    @functools.partial(
        pl.kernel,
        out_type=jax.ShapeDtypeStruct((NC, n_nodes, d_feat), jnp.float32),
        mesh=mesh,
        compiler_params=pltpu.CompilerParams(use_tc_tiling_on_sc=False),
        scratch_types=[
            pltpu.VMEM((CH,), jnp.int32),
            pltpu.VMEM((CH,), jnp.int32),
            pltpu.VMEM((CH,), jnp.int32),
            pltpu.VMEM((CH,), jnp.int32),
            pltpu.VMEM((CH,), jnp.int32),
            pltpu.VMEM((CH,), jnp.int32),
            pltpu.VMEM((CH, half), jnp.float32),
            pltpu.VMEM((CH, half), jnp.float32),
            pltpu.VMEM((CH, d_feat), jnp.float32),
            pltpu.VMEM((CH, d_feat), jnp.float32),
            pltpu.VMEM_SHARED((nacc, d_feat), jnp.float32),
            pltpu.SemaphoreType.DMA,
            pltpu.SemaphoreType.DMA,
            pltpu.SemaphoreType.DMA,
        ],
    )
    def sc_kernel(xr, src, dst, out, src_v0, src_v1, dst_v0, dst_v1,
                  idx_v0, idx_v1, gbuf0, gbuf1, sbuf0, sbuf1, acc,
                  isem, gsem, ssem):
        c = lax.axis_index("c")
        s = lax.axis_index("s")
        src_v = (src_v0, src_v1)
        dst_v = (dst_v0, dst_v1)
        idx_v = (idx_v0, idx_v1)
        gbuf = (gbuf0, gbuf1)
        sbuf = (sbuf0, sbuf1)

        def zrow(r, _):
            for j in range(d_feat // LANES):
                sbuf0[r, pl.ds(j * LANES, LANES)] = jnp.zeros(
                    (LANES,), jnp.float32)
            return 0
        lax.fori_loop(0, CH, zrow, 0)
        zbase = s * rpw
        for off, step in _chunked(rpw):
            pltpu.sync_copy(sbuf0.at[pl.ds(0, step)],
                            acc.at[pl.ds(zbase + off, step)])
        if ztail:
            @pl.when(s == NS - 1)
            def _():
                for off, step in _chunked(ztail):
                    pltpu.sync_copy(
                        sbuf0.at[pl.ds(0, step)],
                        acc.at[pl.ds(NS * rpw + off, step)])
        plsc.subcore_barrier()

        ebase = s * epw

        def compute_idx(b):
            for i in range(CH // LANES):
                v = src_v[b][pl.ds(i * LANES, LANES)]
                idx_v[b][pl.ds(i * LANES, LANES)] = v * 2 + c

        def compute_chunk(b):
            def edge_body(e2, _):
                e0 = e2 * 4
                uj = [(u, j) for u in range(4)
                      for j in range(half // LANES)]
                vs = [gbuf[b][e0 + u, pl.ds(j * LANES, LANES)]
                      for (u, j) in uj]
                gs = [jnp.maximum(v, 0.0) + EPS for v in vs]
                exs = [jnp.exp(g) for g in gs]
                gexs = [g * ex for g, ex in zip(gs, exs)]
                for (u, j), ex, gex in zip(uj, exs, gexs):
                    sbuf[b][e0 + u, pl.ds(j * LANES, LANES)] = ex
                    sbuf[b][e0 + u, pl.ds(half + j * LANES, LANES)] = gex
                return 0
            lax.fori_loop(0, CH // 4, edge_body, 0)

        pltpu.sync_copy(src.at[pl.ds(ebase, CH)], src_v0)
        pltpu.sync_copy(dst.at[pl.ds(ebase, CH)], dst_v0)
        compute_idx(0)
        pltpu.async_copy(xr.at[idx_v0], gbuf0, gsem)

        def sub_iter(k, a, b):
            @pl.when(k >= 1)
            def _():
                pltpu.make_async_copy(
                    sbuf[b], acc.at[dst_v[b]], ssem).wait()
            nb = ebase + (k + 1) * CH
            d_src = pltpu.make_async_copy(src.at[pl.ds(nb, CH)],
                                          src_v[b], isem)
            d_dst = pltpu.make_async_copy(dst.at[pl.ds(nb, CH)],
                                          dst_v[b], isem)

            @pl.when(k + 1 < nchunk)
            def _():
                d_src.start()
                d_dst.start()
            pltpu.make_async_copy(xr.at[idx_v[a]], gbuf[a], gsem).wait()
            compute_chunk(a)
            pltpu.async_copy(sbuf[a], acc.at[dst_v[a]], ssem, add=True)

            @pl.when(k + 1 < nchunk)
            def _():
                d_src.wait()
                d_dst.wait()
                compute_idx(b)
                pltpu.async_copy(xr.at[idx_v[b]], gbuf[b], gsem)

        def loop_body(k2, _):
            sub_iter(k2 * 2, 0, 1)
            sub_iter(k2 * 2 + 1, 1, 0)
            return 0
        lax.fori_loop(0, nchunk // 2, loop_body, 0)
        pltpu.make_async_copy(sbuf1, acc.at[dst_v1], ssem).wait()
        plsc.subcore_barrier()

        obase = s * rpw
        for off, step in _chunked(rpw):
            pltpu.sync_copy(acc.at[pl.ds(obase + off, step)],
                            out.at[c, pl.ds(obase + off, step)])
        if tail:
            @pl.when(s == NS - 1)
            def _():
                for off, step in _chunked(tail):
                    pltpu.sync_copy(
                        acc.at[pl.ds(NS * rpw + off, step)],
                        out.at[c, pl.ds(NS * rpw + off, step)])

    return sc_kernel


def _tc_mlp1(n_nodes, d_feat, h_feat, blk):
    half = d_feat // 2
    grid = n_nodes // blk

    def body(x_ref, s_ref, w1_ref, b1_ref, h1_ref, sums_ref):
        s0 = s_ref[0]
        s1 = s_ref[1]
        ex = jnp.concatenate([s0[:, :half], s1[:, :half]], axis=1)
        gex = jnp.concatenate([s0[:, half:], s1[:, half:]], axis=1)
        m = gex / (ex + 1e-16)
        h = x_ref[...] + m
        h1 = jnp.dot(h, w1_ref[...],
                     preferred_element_type=jnp.float32) + b1_ref[...]
        h1_ref[...] = h1

        @pl.when(pl.program_id(0) == 0)
        def _():
            sums_ref[...] = jnp.zeros_like(sums_ref)

        upd = jnp.concatenate(
            [jnp.sum(h1, axis=0, keepdims=True),
             jnp.sum(h1 * h1, axis=0, keepdims=True),
             jnp.zeros((6, h_feat), jnp.float32)], axis=0)
        sums_ref[...] += upd

    return pl.pallas_call(
        body,
        grid=(grid,),
        in_specs=[
            pl.BlockSpec((blk, d_feat), lambda i: (i, 0)),
            pl.BlockSpec((NC, blk, d_feat), lambda i: (0, i, 0)),
            pl.BlockSpec((d_feat, h_feat), lambda i: (0, 0)),
            pl.BlockSpec((1, h_feat), lambda i: (0, 0)),
        ],
        out_specs=[
            pl.BlockSpec((blk, h_feat), lambda i: (i, 0)),
            pl.BlockSpec((8, h_feat), lambda i: (0, 0)),
        ],
        out_shape=[
            jax.ShapeDtypeStruct((n_nodes, h_feat), jnp.float32),
            jax.ShapeDtypeStruct((8, h_feat), jnp.float32),
        ],
    )


def _tc_mlp2(n_nodes, d_feat, h_feat, blk):
    grid = n_nodes // blk
    inv_n = 1.0 / n_nodes

    def body(h1_ref, sums_ref, gamma_ref, beta_ref, w2_ref, b2_ref, o_ref):
        mean = sums_ref[0:1, :] * inv_n
        var = sums_ref[1:2, :] * inv_n - mean * mean
        scale = lax.rsqrt(var + 1e-05) * gamma_ref[...]
        h1n = (h1_ref[...] - mean) * scale + beta_ref[...]
        h1n = jnp.maximum(h1n, 0.0)
        o_ref[...] = jnp.dot(h1n, w2_ref[...],
                             preferred_element_type=jnp.float32) + b2_ref[...]

    return pl.pallas_call(
        body,
        grid=(grid,),
        in_specs=[
            pl.BlockSpec((blk, h_feat), lambda i: (i, 0)),
            pl.BlockSpec((8, h_feat), lambda i: (0, 0)),
            pl.BlockSpec((1, h_feat), lambda i: (0, 0)),
            pl.BlockSpec((1, h_feat), lambda i: (0, 0)),
            pl.BlockSpec((h_feat, d_feat), lambda i: (0, 0)),
            pl.BlockSpec((1, d_feat), lambda i: (0, 0)),
        ],
        out_specs=pl.BlockSpec((blk, d_feat), lambda i: (i, 0)),
        out_shape=jax.ShapeDtypeStruct((n_nodes, d_feat), jnp.float32),
    )


def kernel(x, edge_index, W1, b1, gamma, beta, W2, b2):
    n, d = x.shape
    h_feat = W1.shape[1]
    e = edge_index.shape[1]

    e_pad = ((e + 2 * NS * CH - 1) // (2 * NS * CH)) * (2 * NS * CH)
    pad = e_pad - e
    src = edge_index[0]
    dst = edge_index[1]
    if pad:
        src = jnp.concatenate([src, jnp.zeros((pad,), jnp.int32)])
        dst = jnp.concatenate([dst, jnp.full((pad,), n, jnp.int32)])
    xr = x.reshape(2 * n, d // 2)

    s_acc = _sc_edge_pass(n, d, e_pad)(xr, src, dst)

    blk = 1000 if n % 1000 == 0 else n // 8
    h1, sums = _tc_mlp1(n, d, h_feat, blk)(
        x, s_acc, W1, b1.reshape(1, h_feat))
    out = _tc_mlp2(n, d, h_feat, blk)(
        h1, sums, gamma.reshape(1, h_feat), beta.reshape(1, h_feat),
        W2, b2.reshape(1, d))
    return out


# speedup vs baseline: 15.5724x; 1.4339x over previous
import functools

import jax
import jax.numpy as jnp
from jax import lax
from jax.experimental import pallas as pl
from jax.experimental.pallas import tpu as pltpu
from jax.experimental.pallas import tpu_sc as plsc

EPS = 1e-07
NC = 2
NS = 16
LANES = 16
CH = 128


def _sc_edge_pass(n_nodes, d_feat, e_pad):
    half = d_feat // 2
    epw = e_pad // NS
    nchunk = epw // CH
    nacc = n_nodes + LANES
    rpw = (n_nodes // NS) & ~7
    tail = n_nodes - NS * rpw
    ztail = nacc - NS * rpw
    mesh = plsc.VectorSubcoreMesh(core_axis_name="c", subcore_axis_name="s")

    def _chunked(total):
        done = 0
        while done < total:
            step = min(CH, total - done)
            yield done, step
            done += step

    @functools.partial(
        pl.kernel,
        out_type=jax.ShapeDtypeStruct((NC, n_nodes, half), jnp.float32),
        mesh=mesh,
        compiler_params=pltpu.CompilerParams(use_tc_tiling_on_sc=False,
                                             needs_layout_passes=False),
        scratch_types=[
            pltpu.VMEM((nchunk, CH), jnp.int32),
            pltpu.VMEM((nchunk, CH), jnp.int32),
            pltpu.VMEM((CH, half), jnp.float32),
            pltpu.VMEM((CH, half), jnp.float32),
            pltpu.VMEM((CH, d_feat), jnp.bfloat16),
            pltpu.VMEM((CH, d_feat), jnp.bfloat16),
            pltpu.VMEM_SHARED((nacc, d_feat), jnp.bfloat16),
            pltpu.SemaphoreType.DMA,
            pltpu.SemaphoreType.DMA,
        ],
    )
    def sc_kernel(xr, src, dst, out, idx_big, dst_big,
                  gbuf0, gbuf1, sbuf0, sbuf1, acc, gsem, ssem):
        c = lax.axis_index("c")
        s = lax.axis_index("s")
        gbuf = (gbuf0, gbuf1)
        sbuf = (sbuf0, sbuf1)
        pk = 2 * LANES

        def zrow(r, _):
            for j in range(d_feat // pk):
                sbuf0[r, pl.ds(j * pk, pk)] = jnp.zeros(
                    (pk,), jnp.bfloat16)
            return 0
        lax.fori_loop(0, CH, zrow, 0)
        zbase = s * rpw
        for off, step in _chunked(rpw):
            pltpu.sync_copy(sbuf0.at[pl.ds(0, step)],
                            acc.at[pl.ds(zbase + off, step)])
        if ztail:
            @pl.when(s == NS - 1)
            def _():
                for off, step in _chunked(ztail):
                    pltpu.sync_copy(
                        sbuf0.at[pl.ds(0, step)],
                        acc.at[pl.ds(NS * rpw + off, step)])
        plsc.subcore_barrier()

        pltpu.sync_copy(src.at[pl.ds(s * nchunk, nchunk)], idx_big)
        pltpu.sync_copy(dst.at[pl.ds(s * nchunk, nchunk)], dst_big)

        def idx_row(r, _):
            vs = [idx_big[r, pl.ds(i * LANES, LANES)]
                  for i in range(CH // LANES)]
            ws = [v * 2 + c for v in vs]
            for i, w in enumerate(ws):
                idx_big[r, pl.ds(i * LANES, LANES)] = w
            return 0
        lax.fori_loop(0, nchunk, idx_row, 0)

        def compute_chunk(b):
            def edge_body(e2, _):
                e0 = e2 * 4
                uj = [(u, j) for u in range(4)
                      for j in range(half // LANES)]
                vs = [gbuf[b][e0 + u, pl.ds(j * LANES, LANES)]
                      for (u, j) in uj]
                gs = [jnp.maximum(v, 0.0) + EPS for v in vs]
                exs = [jnp.exp(g) for g in gs]
                gexs = [g * ex for g, ex in zip(gs, exs)]
                pks = [plsc.pack(ex, gex, format=plsc.PackFormat.INTERLEAVED)
                       for ex, gex in zip(exs, gexs)]
                for (u, j), pv in zip(uj, pks):
                    sbuf[b][e0 + u, pl.ds(j * pk, pk)] = pv
                return 0
            lax.fori_loop(0, CH // 4, edge_body, 0)

        pltpu.async_copy(xr.at[idx_big.at[0]], gbuf0, gsem)
        pltpu.async_copy(xr.at[idx_big.at[1]], gbuf1, gsem)

        def sub_iter(k, a, b):
            pltpu.make_async_copy(xr.at[idx_big.at[k]], gbuf[a],
                                  gsem).wait()
            compute_chunk(a)

            @pl.when(k >= 1)
            def _():
                pltpu.make_async_copy(
                    sbuf[b], acc.at[dst_big.at[k - 1]], ssem).wait()
            pltpu.async_copy(sbuf[a], acc.at[dst_big.at[k]], ssem,
                             add=True)

            @pl.when(k + 2 < nchunk)
            def _():
                pltpu.async_copy(xr.at[idx_big.at[k + 2]], gbuf[a], gsem)

        def loop_body(k2, _):
            sub_iter(k2 * 2, 0, 1)
            sub_iter(k2 * 2 + 1, 1, 0)
            return 0
        lax.fori_loop(0, nchunk // 2, loop_body, 0)
        pltpu.make_async_copy(
            sbuf1, acc.at[dst_big.at[nchunk - 1]], ssem).wait()
        plsc.subcore_barrier()

        def emit_rows(base_row, step):
            pltpu.sync_copy(acc.at[pl.ds(base_row, step)],
                            sbuf0.at[pl.ds(0, step)])

            def rbody(r2, _):
                r0 = r2 * 2
                uj = [(u, j) for u in range(2)
                      for j in range(d_feat // pk)]
                pvs = [sbuf0[r0 + u, pl.ds(j * pk, pk)] for (u, j) in uj]
                egs = [plsc.unpack(pv, format=plsc.PackFormat.INTERLEAVED,
                                   preferred_element_type=jnp.float32)
                       for pv in pvs]
                ms = [gex / (ex + 1e-16) for ex, gex in egs]
                for (u, j), m in zip(uj, ms):
                    gbuf0[r0 + u, pl.ds(j * LANES, LANES)] = m
                return 0
            lax.fori_loop(0, step // 2, rbody, 0)
            pltpu.sync_copy(gbuf0.at[pl.ds(0, step)],
                            out.at[c, pl.ds(base_row, step)])

        obase = s * rpw
        for off, step in _chunked(rpw):
            emit_rows(obase + off, step)
        if tail:
            @pl.when(s == NS - 1)
            def _():
                for off, step in _chunked(tail):
                    emit_rows(NS * rpw + off, step)

    return sc_kernel


def _tc_mlp1(n_nodes, d_feat, h_feat, blk):
    half = d_feat // 2
    grid = n_nodes // blk

    def body(x_ref, s_ref, w1_ref, b1_ref, h1_ref, sums_ref):
        m = jnp.concatenate([s_ref[0], s_ref[1]], axis=1)
        h = x_ref[...] + m
        h1 = jnp.dot(h, w1_ref[...],
                     preferred_element_type=jnp.float32) + b1_ref[...]
        h1_ref[...] = h1

        @pl.when(pl.program_id(0) == 0)
        def _():
            sums_ref[...] = jnp.zeros_like(sums_ref)

        upd = jnp.concatenate(
            [jnp.sum(h1, axis=0, keepdims=True),
             jnp.sum(h1 * h1, axis=0, keepdims=True),
             jnp.zeros((6, h_feat), jnp.float32)], axis=0)
        sums_ref[...] += upd

    return pl.pallas_call(
        body,
        grid=(grid,),
        in_specs=[
            pl.BlockSpec((blk, d_feat), lambda i: (i, 0)),
            pl.BlockSpec((NC, blk, half), lambda i: (0, i, 0)),
            pl.BlockSpec((d_feat, h_feat), lambda i: (0, 0)),
            pl.BlockSpec((1, h_feat), lambda i: (0, 0)),
        ],
        out_specs=[
            pl.BlockSpec((blk, h_feat), lambda i: (i, 0)),
            pl.BlockSpec((8, h_feat), lambda i: (0, 0)),
        ],
        out_shape=[
            jax.ShapeDtypeStruct((n_nodes, h_feat), jnp.float32),
            jax.ShapeDtypeStruct((8, h_feat), jnp.float32),
        ],
    )


def _tc_mlp2(n_nodes, d_feat, h_feat, blk):
    grid = n_nodes // blk
    inv_n = 1.0 / n_nodes

    def body(h1_ref, sums_ref, gamma_ref, beta_ref, w2_ref, b2_ref, o_ref):
        mean = sums_ref[0:1, :] * inv_n
        var = sums_ref[1:2, :] * inv_n - mean * mean
        scale = lax.rsqrt(var + 1e-05) * gamma_ref[...]
        h1n = (h1_ref[...] - mean) * scale + beta_ref[...]
        h1n = jnp.maximum(h1n, 0.0)
        o_ref[...] = jnp.dot(h1n, w2_ref[...],
                             preferred_element_type=jnp.float32) + b2_ref[...]

    return pl.pallas_call(
        body,
        grid=(grid,),
        in_specs=[
            pl.BlockSpec((blk, h_feat), lambda i: (i, 0)),
            pl.BlockSpec((8, h_feat), lambda i: (0, 0)),
            pl.BlockSpec((1, h_feat), lambda i: (0, 0)),
            pl.BlockSpec((1, h_feat), lambda i: (0, 0)),
            pl.BlockSpec((h_feat, d_feat), lambda i: (0, 0)),
            pl.BlockSpec((1, d_feat), lambda i: (0, 0)),
        ],
        out_specs=pl.BlockSpec((blk, d_feat), lambda i: (i, 0)),
        out_shape=jax.ShapeDtypeStruct((n_nodes, d_feat), jnp.float32),
    )


def kernel(x, edge_index, W1, b1, gamma, beta, W2, b2):
    n, d = x.shape
    h_feat = W1.shape[1]
    e = edge_index.shape[1]

    e_pad = ((e + 2 * NS * CH - 1) // (2 * NS * CH)) * (2 * NS * CH)
    pad = e_pad - e
    src = edge_index[0]
    dst = edge_index[1]
    if pad:
        src = jnp.concatenate([src, jnp.zeros((pad,), jnp.int32)])
        dst = jnp.concatenate([dst, jnp.full((pad,), n, jnp.int32)])
    xr = x.reshape(2 * n, d // 2)

    s_acc = _sc_edge_pass(n, d, e_pad)(
        xr, src.reshape(e_pad // CH, CH), dst.reshape(e_pad // CH, CH))

    blk = 1000 if n % 1000 == 0 else n // 8
    h1, sums = _tc_mlp1(n, d, h_feat, blk)(
        x, s_acc, W1, b1.reshape(1, h_feat))
    out = _tc_mlp2(n, d, h_feat, blk)(
        h1, sums, gamma.reshape(1, h_feat), beta.reshape(1, h_feat),
        W2, b2.reshape(1, d))
    return out
